# Initial kernel scaffold; baseline (speedup 1.0000x reference)
import jax
import jax.numpy as jnp
from jax.experimental import pallas as pl


def kernel(x, edge_index, W_g1, b_g1, W_g2, b_g2, W_m1, b_m1, W_m2, b_m2, W_f, b_f, alpha):
    raise NotImplementedError("write your pallas kernel here")


# speedup vs baseline: 15.6080x; 15.6080x over previous
import functools

import jax
import jax.numpy as jnp
from jax import lax
from jax.experimental import pallas as pl
from jax.experimental.pallas import tpu as pltpu
from jax.experimental.pallas import tpu_sc as plsc

_N = 10000
_E = 320000
_F = 128
_NC = 2
_NS = 16
_NW = _NC * _NS
_EPW = _E // _NW
_CHUNK = 128
_NFULL = _EPW // _CHUNK
_TAIL = _EPW - _NFULL * _CHUNK
_N_PAD = 10240
_RPT = _N_PAD // _NS
_LAST = _N - (_NS - 1) * _RPT
_DEG_PAD = _N_PAD
_DPT = _DEG_PAD // _NS

_MESH = plsc.VectorSubcoreMesh(core_axis_name="c", subcore_axis_name="s")
_HIGH = lax.Precision.HIGHEST


def _deg_call(dst, ones_c, z_deg):
    @functools.partial(
        pl.kernel,
        out_type=jax.ShapeDtypeStruct((_NC, _DEG_PAD), jnp.float32),
        mesh=_MESH,
        scratch_types=[
            pltpu.VMEM((_CHUNK,), jnp.int32),
            pltpu.VMEM((_TAIL,), jnp.int32),
            pltpu.VMEM((_CHUNK,), jnp.float32),
            pltpu.VMEM_SHARED((_DEG_PAD,), jnp.float32),
        ],
    )
    def k(dst_hbm, ones_hbm, z_hbm, out_hbm, idx_v, idxt_v, ones_v, acc):
        c = lax.axis_index("c")
        s = lax.axis_index("s")
        wid = s * _NC + c
        pltpu.sync_copy(ones_hbm, ones_v)
        pltpu.sync_copy(z_hbm.at[pl.ds(s * _DPT, _DPT)],
                        acc.at[pl.ds(s * _DPT, _DPT)])
        plsc.subcore_barrier()
        base = wid * _EPW

        @pl.loop(0, _NFULL)
        def _(kk):
            pltpu.sync_copy(dst_hbm.at[pl.ds(base + kk * _CHUNK, _CHUNK)], idx_v)
            pltpu.sync_copy(ones_v, acc.at[idx_v], add=True)

        pltpu.sync_copy(dst_hbm.at[pl.ds(base + _NFULL * _CHUNK, _TAIL)], idxt_v)
        pltpu.sync_copy(ones_v.at[pl.ds(0, _TAIL)], acc.at[idxt_v], add=True)
        plsc.subcore_barrier()
        pltpu.sync_copy(acc.at[pl.ds(s * _DPT, _DPT)],
                        out_hbm.at[c, pl.ds(s * _DPT, _DPT)])

    return k(dst, ones_c, z_deg)


def _agg_call(y, src, dst, z_rows):
    @functools.partial(
        pl.kernel,
        out_type=jax.ShapeDtypeStruct((_NC, _N_PAD, _F), jnp.float32),
        mesh=_MESH,
        scratch_types=[
            pltpu.VMEM((_CHUNK,), jnp.int32),
            pltpu.VMEM((_CHUNK,), jnp.int32),
            pltpu.VMEM((_TAIL,), jnp.int32),
            pltpu.VMEM((_TAIL,), jnp.int32),
            pltpu.VMEM((_CHUNK, _F), jnp.float32),
            pltpu.VMEM_SHARED((_N_PAD, _F), jnp.float32),
            pltpu.SemaphoreType.DMA,
        ],
    )
    def k(y_hbm, src_hbm, dst_hbm, z_hbm, out_hbm,
          src_v, dst_v, srct_v, dstt_v, rows_v, acc, sem):
        c = lax.axis_index("c")
        s = lax.axis_index("s")
        wid = s * _NC + c
        r0 = s * _RPT

        @pl.when(c == 0)
        def _():
            @pl.when(s < _NS - 1)
            def _():
                pltpu.sync_copy(y_hbm.at[pl.ds(r0, _RPT)],
                                acc.at[pl.ds(r0, _RPT)])

            @pl.when(s == _NS - 1)
            def _():
                pltpu.sync_copy(y_hbm.at[pl.ds(r0, _LAST)],
                                acc.at[pl.ds(r0, _LAST)])

        @pl.when(c != 0)
        def _():
            @pl.when(s < _NS - 1)
            def _():
                pltpu.sync_copy(z_hbm.at[pl.ds(r0, _RPT)],
                                acc.at[pl.ds(r0, _RPT)])

            @pl.when(s == _NS - 1)
            def _():
                pltpu.sync_copy(z_hbm.at[pl.ds(r0, _LAST)],
                                acc.at[pl.ds(r0, _LAST)])

        plsc.subcore_barrier()
        base = wid * _EPW

        @pl.loop(0, _NFULL)
        def _(kk):
            off = base + kk * _CHUNK
            pltpu.sync_copy(src_hbm.at[pl.ds(off, _CHUNK)], src_v)
            pltpu.sync_copy(dst_hbm.at[pl.ds(off, _CHUNK)], dst_v)
            pltpu.async_copy(y_hbm.at[src_v], rows_v, sem).wait()
            pltpu.sync_copy(rows_v, acc.at[dst_v], add=True)

        offt = base + _NFULL * _CHUNK
        pltpu.sync_copy(src_hbm.at[pl.ds(offt, _TAIL)], srct_v)
        pltpu.sync_copy(dst_hbm.at[pl.ds(offt, _TAIL)], dstt_v)
        pltpu.async_copy(y_hbm.at[srct_v], rows_v.at[pl.ds(0, _TAIL)], sem).wait()
        pltpu.sync_copy(rows_v.at[pl.ds(0, _TAIL)], acc.at[dstt_v], add=True)

        plsc.subcore_barrier()
        pltpu.sync_copy(acc.at[pl.ds(r0, _RPT)], out_hbm.at[c, pl.ds(r0, _RPT)])

    return k(y, src, dst, z_rows)


_RBLK = 2000
_GRID = _N // _RBLK

_row_spec = pl.BlockSpec((_RBLK, _F), lambda i: (i, 0))
_col_spec = pl.BlockSpec((_RBLK, 1), lambda i: (i, 0))
_w_spec = pl.BlockSpec((_F, _F), lambda i: (0, 0))
_b_spec = pl.BlockSpec((_F,), lambda i: (0,))


def _tc1_call(x, deg0, deg1, W_g1, W_m1, b_m1, W_m2, b_m2):
    def body(x_ref, d0_ref, d1_ref, wg1_ref, wm1_ref, bm1_ref, wm2_ref, bm2_ref,
             y1_ref, m_ref, dinv_ref):
        d = d0_ref[...] + d1_ref[...]
        dinv = lax.rsqrt(d + 1.0)
        xv = x_ref[...]
        xw = jnp.dot(xv, wg1_ref[...], precision=_HIGH,
                     preferred_element_type=jnp.float32)
        y1_ref[...] = xw * dinv
        h = jnp.maximum(
            jnp.dot(xv, wm1_ref[...], precision=_HIGH,
                    preferred_element_type=jnp.float32) + bm1_ref[...], 0.0)
        m_ref[...] = jnp.dot(h, wm2_ref[...], precision=_HIGH,
                             preferred_element_type=jnp.float32) + bm2_ref[...]
        dinv_ref[...] = dinv

    return pl.pallas_call(
        body,
        grid=(_GRID,),
        in_specs=[_row_spec, _col_spec, _col_spec, _w_spec, _w_spec, _b_spec,
                  _w_spec, _b_spec],
        out_specs=(_row_spec, _row_spec, _col_spec),
        out_shape=(
            jax.ShapeDtypeStruct((_N, _F), jnp.float32),
            jax.ShapeDtypeStruct((_N, _F), jnp.float32),
            jax.ShapeDtypeStruct((_N, 1), jnp.float32),
        ),
    )(x, deg0, deg1, W_g1, W_m1, b_m1, W_m2, b_m2)


def _tc2_call(agg0, agg1, dinv, b_g1, W_g2):
    def body(a0_ref, a1_ref, dinv_ref, bg1_ref, wg2_ref, y2_ref):
        dinv = dinv_ref[...]
        g1 = jnp.maximum((a0_ref[...] + a1_ref[...]) * dinv + bg1_ref[...], 0.0)
        y2_ref[...] = jnp.dot(g1, wg2_ref[...], precision=_HIGH,
                              preferred_element_type=jnp.float32) * dinv

    return pl.pallas_call(
        body,
        grid=(_GRID,),
        in_specs=[_row_spec, _row_spec, _col_spec, _b_spec, _w_spec],
        out_specs=_row_spec,
        out_shape=jax.ShapeDtypeStruct((_N, _F), jnp.float32),
    )(agg0, agg1, dinv, b_g1, W_g2)


def _tc3_call(agg0, agg1, dinv, b_g2, m, W_f, b_f, alpha11):
    def body(a0_ref, a1_ref, dinv_ref, bg2_ref, m_ref, wf_ref, bf_ref, al_ref,
             out_ref):
        g2 = (a0_ref[...] + a1_ref[...]) * dinv_ref[...] + bg2_ref[...]
        wf = wf_ref[...]
        a = jnp.clip(al_ref[...], 0.0, 1.0)
        gt = jnp.dot(g2, wf[:_F], precision=_HIGH,
                     preferred_element_type=jnp.float32)
        mt = jnp.dot(m_ref[...], wf[_F:], precision=_HIGH,
                     preferred_element_type=jnp.float32)
        out_ref[...] = a * gt + (1.0 - a) * mt + bf_ref[...]

    return pl.pallas_call(
        body,
        grid=(_GRID,),
        in_specs=[_row_spec, _row_spec, _col_spec, _b_spec, _row_spec,
                  pl.BlockSpec((2 * _F, _F), lambda i: (0, 0)),
                  _b_spec,
                  pl.BlockSpec((1, 1), lambda i: (0, 0))],
        out_specs=_row_spec,
        out_shape=jax.ShapeDtypeStruct((_N, _F), jnp.float32),
    )(agg0, agg1, dinv, b_g2, m, W_f, b_f, alpha11)


def kernel(x, edge_index, W_g1, b_g1, W_g2, b_g2, W_m1, b_m1, W_m2, b_m2,
           W_f, b_f, alpha):
    src = edge_index[0]
    dst = edge_index[1]
    ones_c = jnp.ones((_CHUNK,), jnp.float32)
    z_deg = jnp.zeros((_DEG_PAD,), jnp.float32)
    z_rows = jnp.zeros((_N, _F), jnp.float32)
    alpha11 = jnp.reshape(alpha, (1, 1)).astype(jnp.float32)

    degp = _deg_call(dst, ones_c, z_deg)
    deg0 = degp[0].reshape(_DEG_PAD, 1)
    deg1 = degp[1].reshape(_DEG_PAD, 1)
    y1, m, dinv = _tc1_call(x, deg0, deg1, W_g1, W_m1, b_m1, W_m2, b_m2)
    aggp = _agg_call(y1, src, dst, z_rows)
    y2 = _tc2_call(aggp[0, :_N], aggp[1, :_N], dinv, b_g1, W_g2)
    aggp2 = _agg_call(y2, src, dst, z_rows)
    out = _tc3_call(aggp2[0, :_N], aggp2[1, :_N], dinv, b_g2, m, W_f, b_f,
                    alpha11)
    return out
